# baseline (speedup 1.0000x reference)
import functools

import jax
import jax.numpy as jnp
from jax import lax
from jax.experimental import pallas as pl
from jax.experimental.pallas import tpu as pltpu
from jax.experimental.pallas import tpu_sc as plsc

CH = 128
NTILES = 16
NCORES = 2
LANES = 16


def _mm_body(x_ref, w_ref, o_ref):
    o_ref[...] = jnp.dot(x_ref[...], w_ref[...],
                         preferred_element_type=jnp.float32)


def _matmul(x2, W):
    M, D = x2.shape
    H = W.shape[1]
    BM = 2000
    return pl.pallas_call(
        _mm_body,
        grid=(M // BM,),
        in_specs=[pl.BlockSpec((BM, D), lambda i: (i, 0)),
                  pl.BlockSpec((D, H), lambda i: (0, 0))],
        out_specs=pl.BlockSpec((BM, H), lambda i: (i, 0)),
        out_shape=jax.ShapeDtypeStruct((M, H), jnp.float32),
    )(x2, W)


SUP = 16


def _sc_body(Npad, D, nch_t, batches_per_core,
             xw_hbm, idx_hbm, val_hbm, out_hbm,
             acc, rb0, rb1, islab, vslab, gs0, gs1):
    cid = lax.axis_index("c")
    sid = lax.axis_index("s")
    rows_per_tile = Npad // NTILES
    zrows = rb0.shape[0]
    nzcopy = rows_per_tile // zrows
    groups = D // LANES
    nsup = nch_t // SUP
    egroups = CH // LANES

    def scale(rb, k):
        def grp(g, _):
            vv = vslab[k, pl.ds(g * LANES, LANES)]
            for l in range(LANES):
                sc = jnp.broadcast_to(vv[l:l + 1], (LANES,))
                e = g * LANES + l
                for j in range(groups):
                    sl = pl.ds(j * LANES, LANES)
                    rb[e, sl] = rb[e, sl] * sc
            return 0

        lax.fori_loop(0, egroups, grp, 0)

    bufs = ((rb0, gs0), (rb1, gs1))

    def batch(bi, _):
        b = cid * batches_per_core + bi
        base_row = sid * rows_per_tile

        zv = jnp.zeros((LANES,), jnp.float32)

        def zrow(i, _):
            for j in range(groups):
                rb0[i, pl.ds(j * LANES, LANES)] = zv
            return 0

        lax.fori_loop(0, zrows, zrow, 0)
        for i in range(nzcopy):
            pltpu.sync_copy(rb0, acc.at[pl.ds(base_row + i * zrows, zrows)])
        plsc.subcore_barrier()

        def superchunk(s, _):
            c0 = sid * nch_t + s * SUP
            pltpu.sync_copy(idx_hbm.at[b, pl.ds(c0, SUP)], islab)
            pltpu.sync_copy(val_hbm.at[b, pl.ds(c0, SUP)], vslab)
            pltpu.async_copy(xw_hbm.at[islab.at[0, 1]], rb0, gs0)
            pltpu.async_copy(xw_hbm.at[islab.at[1, 1]], rb1, gs1)

            def pair(t, _):
                for p, (rb, gs) in enumerate(bufs):
                    k = 2 * t + p
                    pltpu.make_async_copy(
                        xw_hbm.at[islab.at[k, 1]], rb, gs).wait()
                    scale(rb, k)
                    pltpu.sync_copy(rb, acc.at[islab.at[k, 0]], add=True)
                    pltpu.async_copy(xw_hbm.at[islab.at[k + 2, 1]], rb, gs)
                return 0

            lax.fori_loop(0, SUP // 2 - 1, pair, 0)
            for p, (rb, gs) in enumerate(bufs):
                k = SUP - 2 + p
                pltpu.make_async_copy(
                    xw_hbm.at[islab.at[k, 1]], rb, gs).wait()
                scale(rb, k)
                pltpu.sync_copy(rb, acc.at[islab.at[k, 0]], add=True)
            return 0

        lax.fori_loop(0, nsup, superchunk, 0)
        plsc.subcore_barrier()

        for i in range(nzcopy):
            r0 = base_row + i * zrows
            pltpu.sync_copy(acc.at[pl.ds(r0, zrows)],
                            out_hbm.at[b, pl.ds(r0, zrows)])
        plsc.subcore_barrier()
        return 0

    lax.fori_loop(0, batches_per_core, batch, 0)


def _sc_aggregate(xw_flat, idx_arr, val_arr, B, Npad, D, nch_t):
    mesh = plsc.VectorSubcoreMesh(core_axis_name="c", subcore_axis_name="s")
    body = functools.partial(_sc_body, Npad, D, nch_t, B // NCORES)
    return pl.kernel(
        body,
        out_type=jax.ShapeDtypeStruct((B, Npad, D), jnp.float32),
        mesh=mesh,
        scratch_types=[
            pltpu.VMEM_SHARED((Npad, D), jnp.float32),
            pltpu.VMEM((CH, D), jnp.float32),
            pltpu.VMEM((CH, D), jnp.float32),
            pltpu.VMEM((SUP, 2, CH), jnp.int32),
            pltpu.VMEM((SUP, CH), jnp.float32),
            pltpu.SemaphoreType.DMA,
            pltpu.SemaphoreType.DMA,
        ],
    )(xw_flat, idx_arr, val_arr)


def kernel(x, adj_indices, adj_values, W):
    B, N, D = x.shape
    H = W.shape[1]
    E = adj_indices.shape[2]

    per_tile = -(-E // NTILES)
    nch_t = -(-per_tile // CH)
    nch_t = -(-nch_t // SUP) * SUP
    e_pad = NTILES * nch_t * CH

    row = adj_indices[:, 0, :].astype(jnp.int32)
    col = adj_indices[:, 1, :].astype(jnp.int32)
    col, row, val = lax.sort((col, row, adj_values), dimension=1, num_keys=1)
    colg = col + (jnp.arange(B, dtype=jnp.int32) * N)[:, None]
    pad = e_pad - E
    row = jnp.pad(row, ((0, 0), (0, pad)))
    colg = jnp.pad(colg, ((0, 0), (0, pad)))
    val = jnp.pad(val, ((0, 0), (0, pad)))
    idx_arr = jnp.stack(
        [a.reshape(B, NTILES * nch_t, CH) for a in (row, colg)], axis=2)
    val_arr = val.reshape(B, NTILES * nch_t, CH)

    Npad = -(-N // (NTILES * 128)) * (NTILES * 128)
    xw = _matmul(x.reshape(B * N, D), W)
    out = _sc_aggregate(xw, idx_arr, val_arr, B, Npad, H, nch_t)
    return out[:, :N, :]


# speedup vs baseline: 3.1362x; 3.1362x over previous
import functools

import jax
import jax.numpy as jnp
from jax import lax
from jax.experimental import pallas as pl
from jax.experimental.pallas import tpu as pltpu
from jax.experimental.pallas import tpu_sc as plsc

CH = 128
NTILES = 16
NCORES = 2
LANES = 16


def _mm_body(x_ref, w_ref, o_ref):
    o_ref[...] = jnp.dot(x_ref[...], w_ref[...],
                         preferred_element_type=jnp.float32)


def _matmul(x2, W):
    M, D = x2.shape
    H = W.shape[1]
    BM = 2000
    return pl.pallas_call(
        _mm_body,
        grid=(M // BM,),
        in_specs=[pl.BlockSpec((BM, D), lambda i: (i, 0)),
                  pl.BlockSpec((D, H), lambda i: (0, 0))],
        out_specs=pl.BlockSpec((BM, H), lambda i: (i, 0)),
        out_shape=jax.ShapeDtypeStruct((M, H), jnp.float32),
    )(x2, W)


SUP = 32


def _sc_body(Npad, D, nch_t, batches_per_core,
             xw_hbm, idx_hbm, val_hbm, out_hbm,
             acc, rb0, rb1, islab, vslab, gs0, gs1):
    cid = lax.axis_index("c")
    sid = lax.axis_index("s")
    rows_per_tile = Npad // NTILES
    zrows = rb0.shape[0]
    nzcopy = rows_per_tile // zrows
    groups = D // LANES
    nsup = nch_t // SUP
    egroups = CH // LANES

    def scale(rb, k):
        def grp(g, _):
            vv = vslab[k, pl.ds(g * LANES, LANES)]
            for l in range(LANES):
                sc = jnp.broadcast_to(vv[l:l + 1], (LANES,))
                e = g * LANES + l
                for j in range(groups):
                    sl = pl.ds(j * LANES, LANES)
                    rb[e, sl] = rb[e, sl] * sc
            return 0

        lax.fori_loop(0, egroups, grp, 0)

    bufs = ((rb0, gs0), (rb1, gs1))

    def batch(bi, _):
        b = cid * batches_per_core + bi
        base_row = sid * rows_per_tile

        zv = jnp.zeros((LANES,), jnp.float32)

        def zrow(i, _):
            for j in range(groups):
                rb0[i, pl.ds(j * LANES, LANES)] = zv
            return 0

        lax.fori_loop(0, zrows, zrow, 0)
        for i in range(nzcopy):
            pltpu.sync_copy(rb0, acc.at[pl.ds(base_row + i * zrows, zrows)])
        plsc.subcore_barrier()

        def superchunk(s, _):
            c0 = sid * nch_t + s * SUP
            pltpu.sync_copy(idx_hbm.at[b, pl.ds(c0, SUP)], islab)
            pltpu.sync_copy(val_hbm.at[b, pl.ds(c0, SUP)], vslab)
            pltpu.async_copy(xw_hbm.at[islab.at[0, 1]], rb0, gs0)
            pltpu.async_copy(xw_hbm.at[islab.at[1, 1]], rb1, gs1)

            def pair(t, _):
                for p, (rb, gs) in enumerate(bufs):
                    k = 2 * t + p
                    pltpu.make_async_copy(
                        xw_hbm.at[islab.at[k, 1]], rb, gs).wait()
                    scale(rb, k)
                    pltpu.sync_copy(rb, acc.at[islab.at[k, 0]], add=True)
                    pltpu.async_copy(xw_hbm.at[islab.at[k + 2, 1]], rb, gs)
                return 0

            lax.fori_loop(0, SUP // 2 - 1, pair, 0)
            for p, (rb, gs) in enumerate(bufs):
                k = SUP - 2 + p
                pltpu.make_async_copy(
                    xw_hbm.at[islab.at[k, 1]], rb, gs).wait()
                scale(rb, k)
                pltpu.sync_copy(rb, acc.at[islab.at[k, 0]], add=True)
            return 0

        lax.fori_loop(0, nsup, superchunk, 0)
        plsc.subcore_barrier()

        for i in range(nzcopy):
            r0 = base_row + i * zrows
            pltpu.sync_copy(acc.at[pl.ds(r0, zrows)],
                            out_hbm.at[b, pl.ds(r0, zrows)])
        plsc.subcore_barrier()
        return 0

    lax.fori_loop(0, batches_per_core, batch, 0)


def _sc_aggregate(xw_flat, idx_arr, val_arr, B, Npad, D, nch_t):
    mesh = plsc.VectorSubcoreMesh(core_axis_name="c", subcore_axis_name="s")
    body = functools.partial(_sc_body, Npad, D, nch_t, B // NCORES)
    return pl.kernel(
        body,
        out_type=jax.ShapeDtypeStruct((B, Npad, D), jnp.float32),
        mesh=mesh,
        scratch_types=[
            pltpu.VMEM_SHARED((Npad, D), jnp.float32),
            pltpu.VMEM((CH, D), jnp.float32),
            pltpu.VMEM((CH, D), jnp.float32),
            pltpu.VMEM((SUP, 2, CH), jnp.int32),
            pltpu.VMEM((SUP, CH), jnp.float32),
            pltpu.SemaphoreType.DMA,
            pltpu.SemaphoreType.DMA,
        ],
    )(xw_flat, idx_arr, val_arr)


def kernel(x, adj_indices, adj_values, W):
    B, N, D = x.shape
    H = W.shape[1]
    E = adj_indices.shape[2]

    per_tile = -(-E // NTILES)
    nch_t = -(-per_tile // CH)
    nch_t = -(-nch_t // SUP) * SUP
    e_pad = NTILES * nch_t * CH

    row = adj_indices[:, 0, :].astype(jnp.int32)
    col = adj_indices[:, 1, :].astype(jnp.int32)
    val = adj_values
    colg = col + (jnp.arange(B, dtype=jnp.int32) * N)[:, None]
    pad = e_pad - E
    row = jnp.pad(row, ((0, 0), (0, pad)))
    colg = jnp.pad(colg, ((0, 0), (0, pad)))
    val = jnp.pad(val, ((0, 0), (0, pad)))
    idx_arr = jnp.stack(
        [a.reshape(B, NTILES * nch_t, CH) for a in (row, colg)], axis=2)
    val_arr = val.reshape(B, NTILES * nch_t, CH)

    Npad = -(-N // (NTILES * 128)) * (NTILES * 128)
    xw = _matmul(x.reshape(B * N, D), W)
    out = _sc_aggregate(xw, idx_arr, val_arr, B, Npad, H, nch_t)
    return out[:, :N, :]
